# baseline (speedup 1.0000x reference)
import functools

import jax
import jax.numpy as jnp
from jax import lax
from jax.experimental import pallas as pl
from jax.experimental.pallas import tpu as pltpu
from jax.experimental.pallas import tpu_sc as plsc

N = 10000
E = 320000
KEYSPACE = N * N
NC = 2
NS = 16
NW = NC * NS
L = 16
SPAN = KEYSPACE // NW
WORDS = ((SPAN + 31) // 32 + L - 1) // L * L
CHUNK = 4000
ROUNDS = E // CHUNK
IN_VECS = CHUNK // L


def _zero_bitmap(bm):
    zeros = jnp.zeros((L,), jnp.int32)

    def body(i, carry):
        bm[pl.ds(i * L, L)] = zeros
        return carry

    lax.fori_loop(0, WORDS // L, body, 0, unroll=False)


def _scan_stream(edges_hbm, bm, lo, row0, row1):
    def chunk_body(g, cnt):
        base = g * CHUNK
        pltpu.sync_copy(edges_hbm.at[pl.ds(base, CHUNK)], row0)
        pltpu.sync_copy(edges_hbm.at[pl.ds(E + base, CHUNK)], row1)

        def vec_body(i, cnt):
            e0 = row0[pl.ds(i * L, L)]
            e1 = row1[pl.ds(i * L, L)]
            k = e0 * N + e1
            r = k - lo
            inm = plsc.bitcast(r, jnp.uint32) < jnp.uint32(SPAN)
            rs = jnp.where(inm, r, 0)
            w = lax.shift_right_logical(rs, 5)
            bit = lax.shift_left(jnp.int32(1), rs & 31)
            _, uniq = plsc.scan_count(k, mask=inm)
            uniq = uniq & inm
            old = plsc.load_gather(bm, [w], mask=uniq)
            elig = uniq & ((old & bit) == 0)

            def cond(rem):
                return jnp.any(rem)

            def rmw(rem):
                _, sel = plsc.scan_count(w, mask=rem)
                sel = sel & rem
                plsc.addupdate_scatter(bm, [w], bit, mask=sel)
                return rem & jnp.logical_not(sel)

            lax.while_loop(cond, rmw, elig)
            return cnt + jnp.where(elig, 1, 0)

        return lax.fori_loop(0, IN_VECS, vec_body, cnt, unroll=False)

    return lax.fori_loop(0, ROUNDS, chunk_body, jnp.zeros((L,), jnp.int32),
                         unroll=False)


def _sc_counts(edges1, edges2):
    mesh = plsc.VectorSubcoreMesh(
        core_axis_name="c", subcore_axis_name="s", num_cores=NC,
        num_subcores=NS)

    @functools.partial(
        pl.kernel,
        out_type=jax.ShapeDtypeStruct((3, NW, L), jnp.int32),
        mesh=mesh,
        scratch_types=[
            pltpu.VMEM((WORDS,), jnp.int32),
            pltpu.VMEM((CHUNK,), jnp.int32),
            pltpu.VMEM((CHUNK,), jnp.int32),
            pltpu.VMEM((L,), jnp.int32),
        ],
        compiler_params=pltpu.CompilerParams(needs_layout_passes=False),
    )
    def k(e1_hbm, e2_hbm, out_hbm, bm, row0, row1, stage):
        wid = lax.axis_index("s") * NC + lax.axis_index("c")
        lo = wid * SPAN
        _zero_bitmap(bm)
        c1 = _scan_stream(e1_hbm, bm, lo, row0, row1)
        cu = _scan_stream(e2_hbm, bm, lo, row0, row1)
        _zero_bitmap(bm)
        c2 = _scan_stream(e2_hbm, bm, lo, row0, row1)
        for idx, vec in ((0, c1), (1, cu), (2, c2)):
            stage[...] = vec
            pltpu.sync_copy(stage, out_hbm.at[idx, wid])

    return k(edges1, edges2)


def _tc_finish(counts):
    def body(c_ref, o_ref):
        c = c_ref[...].astype(jnp.float32)
        s1 = jnp.sum(c[0])
        su = jnp.sum(c[1])
        s2 = jnp.sum(c[2])
        o_ref[...] = jnp.reshape((s2 - su) / (s1 + su), (1, 1))

    out = pl.pallas_call(
        body,
        out_shape=jax.ShapeDtypeStruct((1, 1), jnp.float32),
    )(counts)
    return out.reshape(())


def kernel(edges1, edges2, num_nodes):
    del num_nodes
    counts = _sc_counts(edges1.reshape(-1), edges2.reshape(-1))
    return _tc_finish(counts)


# speedup vs baseline: 5.6512x; 5.6512x over previous
import functools

import jax
import jax.numpy as jnp
from jax import lax
from jax.experimental import pallas as pl
from jax.experimental.pallas import tpu as pltpu
from jax.experimental.pallas import tpu_sc as plsc

N = 10000
E = 320000
KEYSPACE = N * N
NC = 2
NS = 16
NW = NC * NS
L = 16
SPAN = KEYSPACE // NW
WORDS = ((SPAN + 31) // 32 + L - 1) // L * L
CHUNK = 2000
ROUNDS = E // CHUNK
IN_VECS = CHUNK // L
CAP = 12000


def _zero_bitmap(bm):
    zeros = jnp.zeros((L,), jnp.int32)

    def body(i, carry):
        bm[pl.ds(i * L, L)] = zeros
        return carry

    lax.fori_loop(0, WORDS // L, body, 0, unroll=8)


def _drain(cbuf, bm, cnt_ref, fill):
    lanes = lax.broadcasted_iota(jnp.int32, (L,), 0)

    def body(j, carry):
        valid = lanes < (fill - j * L)
        rs = cbuf[pl.ds(j * L, L)]
        rs = jnp.where(valid, rs, 0)
        w = lax.shift_right_logical(rs, 5)
        bit = lax.shift_left(jnp.int32(1), rs & 31)
        _, uniq = plsc.scan_count(rs, mask=valid)
        uniq = uniq & valid
        old = plsc.load_gather(bm, [w], mask=uniq)
        elig = uniq & ((old & bit) == 0)
        _, sel = plsc.scan_count(w, mask=elig)
        sel = sel & elig
        plsc.addupdate_scatter(bm, [w], bit, mask=sel)
        left = elig & jnp.logical_not(sel)

        @pl.when(jnp.any(left))
        def _():
            def cond(rem):
                return jnp.any(rem)

            def rmw(rem):
                _, s = plsc.scan_count(w, mask=rem)
                s = s & rem
                plsc.addupdate_scatter(bm, [w], bit, mask=s)
                return rem & jnp.logical_not(s)

            lax.while_loop(cond, rmw, left)

        cnt_ref[...] = cnt_ref[...] + jnp.where(elig, 1, 0)
        return carry

    nit = lax.div(fill + (L - 1), L)
    lax.fori_loop(0, nit, body, 0, unroll=False)


def _scan_stream(edges_hbm, bm, cnt_ref, bufs, sems, cbuf, lo):
    def start(g, r0, r1, sem):
        base = g * CHUNK
        pltpu.async_copy(edges_hbm.at[pl.ds(base, CHUNK)], r0, sem)
        pltpu.async_copy(edges_hbm.at[pl.ds(E + base, CHUNK)], r1, sem)

    def wait(r0, r1, sem):
        src = edges_hbm.at[pl.ds(0, CHUNK)]
        pltpu.make_async_copy(src, r0, sem).wait()
        pltpu.make_async_copy(src, r1, sem).wait()

    start(0, bufs[0], bufs[1], sems[0])

    def compact_chunk(r0, r1, fill):
        def vec_body(i, fill):
            e0 = r0[pl.ds(i * L, L)]
            e1 = r1[pl.ds(i * L, L)]
            k = e0 * N + e1
            r = k - lo
            inm = plsc.bitcast(r, jnp.uint32) < jnp.uint32(SPAN)
            rs = jnp.where(inm, r, 0)
            plsc.store_compressed(cbuf.at[pl.ds(fill, L)], rs, mask=inm)
            pc = plsc.all_reduce_population_count(inm)
            return fill + pc[0]

        return lax.fori_loop(0, IN_VECS, vec_body, fill, unroll=5)

    def chunk_pair(gg, fill):
        for b in (0, 1):
            g = 2 * gg + b
            r0, r1, sem = bufs[2 * b], bufs[2 * b + 1], sems[b]
            n0, n1, nsem = bufs[2 - 2 * b], bufs[3 - 2 * b], sems[1 - b]
            wait(r0, r1, sem)

            @pl.when(g + 1 < ROUNDS)
            def _():
                start(g + 1, n0, n1, nsem)

            def no_drain(fill):
                return fill

            def do_drain(fill):
                _drain(cbuf, bm, cnt_ref, fill)
                return jnp.int32(0)

            fill = lax.cond(fill > CAP - CHUNK, do_drain, no_drain, fill)
            fill = compact_chunk(r0, r1, fill)
        return fill

    fill = lax.fori_loop(0, ROUNDS // 2, chunk_pair, jnp.int32(0),
                         unroll=False)
    _drain(cbuf, bm, cnt_ref, fill)


def _sc_counts(edges1, edges2):
    mesh = plsc.VectorSubcoreMesh(
        core_axis_name="c", subcore_axis_name="s", num_cores=NC,
        num_subcores=NS)

    @functools.partial(
        pl.kernel,
        out_type=jax.ShapeDtypeStruct((3, NW, L), jnp.int32),
        mesh=mesh,
        scratch_types=[
            pltpu.VMEM((WORDS,), jnp.int32),
            pltpu.VMEM((CAP + L,), jnp.int32),
            pltpu.VMEM((CHUNK,), jnp.int32),
            pltpu.VMEM((CHUNK,), jnp.int32),
            pltpu.VMEM((CHUNK,), jnp.int32),
            pltpu.VMEM((CHUNK,), jnp.int32),
            pltpu.VMEM((L,), jnp.int32),
            pltpu.VMEM((L,), jnp.int32),
            pltpu.VMEM((L,), jnp.int32),
            pltpu.SemaphoreType.DMA,
            pltpu.SemaphoreType.DMA,
        ],
        compiler_params=pltpu.CompilerParams(needs_layout_passes=False),
    )
    def k(e1_hbm, e2_hbm, out_hbm, bm, cbuf, b0, b1, b2, b3,
          c1_ref, cu_ref, c2_ref, sem0, sem1):
        wid = lax.axis_index("s") * NC + lax.axis_index("c")
        lo = wid * SPAN
        bufs = (b0, b1, b2, b3)
        sems = (sem0, sem1)
        zero = jnp.zeros((L,), jnp.int32)
        c1_ref[...] = zero
        cu_ref[...] = zero
        c2_ref[...] = zero
        _zero_bitmap(bm)
        _scan_stream(e1_hbm, bm, c1_ref, bufs, sems, cbuf, lo)
        _scan_stream(e2_hbm, bm, cu_ref, bufs, sems, cbuf, lo)
        _zero_bitmap(bm)
        _scan_stream(e2_hbm, bm, c2_ref, bufs, sems, cbuf, lo)
        for idx, ref in ((0, c1_ref), (1, cu_ref), (2, c2_ref)):
            pltpu.sync_copy(ref, out_hbm.at[idx, wid])

    return k(edges1, edges2)


def _tc_finish(counts):
    def body(c_ref, o_ref):
        c = c_ref[...].astype(jnp.float32)
        s1 = jnp.sum(c[0])
        su = jnp.sum(c[1])
        s2 = jnp.sum(c[2])
        o_ref[...] = jnp.reshape((s2 - su) / (s1 + su), (1, 1))

    out = pl.pallas_call(
        body,
        out_shape=jax.ShapeDtypeStruct((1, 1), jnp.float32),
    )(counts)
    return out.reshape(())


def kernel(edges1, edges2, num_nodes):
    del num_nodes
    counts = _sc_counts(edges1.reshape(-1), edges2.reshape(-1))
    return _tc_finish(counts)


# speedup vs baseline: 7.3444x; 1.2996x over previous
import functools

import jax
import jax.numpy as jnp
from jax import lax
from jax.experimental import pallas as pl
from jax.experimental.pallas import tpu as pltpu
from jax.experimental.pallas import tpu_sc as plsc

N = 10000
E = 320000
KEYSPACE = N * N
NC = 2
NS = 16
NW = NC * NS
L = 16
SPAN = KEYSPACE // NW
WORDS = ((SPAN + 31) // 32 + L - 1) // L * L
CHUNK = 3200
ROUNDS = E // CHUNK
IN_PAIRS = CHUNK // (2 * L)
CAPH = 6000

KB = 64000
KG = E // KB


def _keys_tc(edges1, edges2):
    def body(a_ref, b_ref, k1_ref, k2_ref):
        k1_ref[...] = a_ref[0:1, :] * N + a_ref[1:2, :]
        k2_ref[...] = b_ref[0:1, :] * N + b_ref[1:2, :]

    k1, k2 = pl.pallas_call(
        body,
        grid=(KG,),
        in_specs=[
            pl.BlockSpec((2, KB), lambda i: (0, i)),
            pl.BlockSpec((2, KB), lambda i: (0, i)),
        ],
        out_specs=[
            pl.BlockSpec((1, KB), lambda i: (0, i)),
            pl.BlockSpec((1, KB), lambda i: (0, i)),
        ],
        out_shape=[
            jax.ShapeDtypeStruct((1, E), jnp.int32),
            jax.ShapeDtypeStruct((1, E), jnp.int32),
        ],
    )(edges1, edges2)
    return k1.reshape(E), k2.reshape(E)


def _zero_bitmap(bm):
    zeros = jnp.zeros((L,), jnp.int32)

    def body(i, carry):
        bm[pl.ds(i * L, L)] = zeros
        return carry

    lax.fori_loop(0, WORDS // L, body, 0, unroll=8)


def _drain(cbuf, bm, cnt_ref, fill):
    lanes = lax.broadcasted_iota(jnp.int32, (L,), 0)

    def body(j, carry):
        valid = lanes < (fill - j * L)
        rs = cbuf[pl.ds(j * L, L)]
        rs = jnp.where(valid, rs, 0)
        w = lax.shift_right_logical(rs, 5)
        bit = lax.shift_left(jnp.int32(1), rs & 31)
        _, uniq = plsc.scan_count(rs, mask=valid)
        uniq = uniq & valid
        old = plsc.load_gather(bm, [w], mask=uniq)
        elig = uniq & ((old & bit) == 0)
        _, sel = plsc.scan_count(w, mask=elig)
        sel = sel & elig
        plsc.addupdate_scatter(bm, [w], bit, mask=sel)
        left = elig & jnp.logical_not(sel)

        @pl.when(jnp.any(left))
        def _():
            def cond(rem):
                return jnp.any(rem)

            def rmw(rem):
                _, s = plsc.scan_count(w, mask=rem)
                s = s & rem
                plsc.addupdate_scatter(bm, [w], bit, mask=s)
                return rem & jnp.logical_not(s)

            lax.while_loop(cond, rmw, left)

        cnt_ref[...] = cnt_ref[...] + jnp.where(elig, 1, 0)
        return carry

    nit = lax.div(fill + (L - 1), L)
    lax.fori_loop(0, nit, body, 0, unroll=False)


def _scan_stream(keys_hbm, bm, cnt_ref, bufs, sems, cbufa, cbufb, lo):
    def start(g, buf, sem):
        pltpu.async_copy(keys_hbm.at[pl.ds(g * CHUNK, CHUNK)], buf, sem)

    def wait(buf, sem):
        src = keys_hbm.at[pl.ds(0, CHUNK)]
        pltpu.make_async_copy(src, buf, sem).wait()

    start(0, bufs[0], sems[0])

    def compact_chunk(buf, fills):
        def pair_body(i, fills):
            fa, fb = fills
            ka = buf[pl.ds((2 * i) * L, L)]
            kb = buf[pl.ds((2 * i + 1) * L, L)]
            ra = ka - lo
            rb = kb - lo
            inma = plsc.bitcast(ra, jnp.uint32) < jnp.uint32(SPAN)
            inmb = plsc.bitcast(rb, jnp.uint32) < jnp.uint32(SPAN)
            plsc.store_compressed(cbufa.at[pl.ds(fa, L)], ra, mask=inma)
            plsc.store_compressed(cbufb.at[pl.ds(fb, L)], rb, mask=inmb)
            pca = plsc.all_reduce_population_count(inma)
            pcb = plsc.all_reduce_population_count(inmb)
            return fa + pca[0], fb + pcb[0]

        return lax.fori_loop(0, IN_PAIRS, pair_body, fills, unroll=4)

    def chunk_pair(gg, fills):
        for b in (0, 1):
            g = 2 * gg + b
            wait(bufs[b], sems[b])

            @pl.when(g + 1 < ROUNDS)
            def _():
                start(g + 1, bufs[1 - b], sems[1 - b])

            def no_drain(fills):
                return fills

            def do_drain(fills):
                _drain(cbufa, bm, cnt_ref, fills[0])
                _drain(cbufb, bm, cnt_ref, fills[1])
                return jnp.int32(0), jnp.int32(0)

            fa, fb = fills
            pred = jnp.maximum(fa, fb) > CAPH - CHUNK // 2
            fills = lax.cond(pred, do_drain, no_drain, fills)
            fills = compact_chunk(bufs[b], fills)
        return fills

    fills = lax.fori_loop(0, ROUNDS // 2, chunk_pair,
                          (jnp.int32(0), jnp.int32(0)), unroll=False)
    _drain(cbufa, bm, cnt_ref, fills[0])
    _drain(cbufb, bm, cnt_ref, fills[1])


def _sc_counts(keys1, keys2):
    mesh = plsc.VectorSubcoreMesh(
        core_axis_name="c", subcore_axis_name="s", num_cores=NC,
        num_subcores=NS)

    @functools.partial(
        pl.kernel,
        out_type=jax.ShapeDtypeStruct((3, NW, L), jnp.int32),
        mesh=mesh,
        scratch_types=[
            pltpu.VMEM((WORDS,), jnp.int32),
            pltpu.VMEM((CAPH + CHUNK // 2 + L,), jnp.int32),
            pltpu.VMEM((CAPH + CHUNK // 2 + L,), jnp.int32),
            pltpu.VMEM((CHUNK,), jnp.int32),
            pltpu.VMEM((CHUNK,), jnp.int32),
            pltpu.VMEM((L,), jnp.int32),
            pltpu.VMEM((L,), jnp.int32),
            pltpu.VMEM((L,), jnp.int32),
            pltpu.SemaphoreType.DMA,
            pltpu.SemaphoreType.DMA,
        ],
        compiler_params=pltpu.CompilerParams(needs_layout_passes=False),
    )
    def k(k1_hbm, k2_hbm, out_hbm, bm, cbufa, cbufb, b0, b1,
          c1_ref, cu_ref, c2_ref, sem0, sem1):
        wid = lax.axis_index("s") * NC + lax.axis_index("c")
        lo = wid * SPAN
        bufs = (b0, b1)
        sems = (sem0, sem1)
        zero = jnp.zeros((L,), jnp.int32)
        c1_ref[...] = zero
        cu_ref[...] = zero
        c2_ref[...] = zero
        _zero_bitmap(bm)
        _scan_stream(k1_hbm, bm, c1_ref, bufs, sems, cbufa, cbufb, lo)
        _scan_stream(k2_hbm, bm, cu_ref, bufs, sems, cbufa, cbufb, lo)
        _zero_bitmap(bm)
        _scan_stream(k2_hbm, bm, c2_ref, bufs, sems, cbufa, cbufb, lo)
        for idx, ref in ((0, c1_ref), (1, cu_ref), (2, c2_ref)):
            pltpu.sync_copy(ref, out_hbm.at[idx, wid])

    return k(keys1, keys2)


def _tc_finish(counts):
    def body(c_ref, o_ref):
        c = c_ref[...].astype(jnp.float32)
        s1 = jnp.sum(c[0])
        su = jnp.sum(c[1])
        s2 = jnp.sum(c[2])
        o_ref[...] = jnp.reshape((s2 - su) / (s1 + su), (1, 1))

    out = pl.pallas_call(
        body,
        out_shape=jax.ShapeDtypeStruct((1, 1), jnp.float32),
    )(counts)
    return out.reshape(())


def kernel(edges1, edges2, num_nodes):
    del num_nodes
    keys1, keys2 = _keys_tc(edges1, edges2)
    counts = _sc_counts(keys1, keys2)
    return _tc_finish(counts)
